# Initial kernel scaffold; baseline (speedup 1.0000x reference)
import jax
import jax.numpy as jnp
from jax.experimental import pallas as pl


def kernel(positions, features, opacities, w2c, cam_position, mode):
    raise NotImplementedError("write your pallas kernel here")


# speedup vs baseline: 1.2310x; 1.2310x over previous
import functools

import jax
import jax.numpy as jnp
from jax import lax
from jax.experimental import pallas as pl
from jax.experimental.pallas import tpu as pltpu
from jax.experimental.pallas import tpu_sc as plsc

W, H = 1280, 720
FX, FY = 1000.0, 1000.0
CX, CY = 640.0, 360.0
NEAR, FAR = 0.1, 100.0

NC, NS = 2, 16
STRIP_ROWS = H // 4
SP = STRIP_ROWS * W
B = 128
NGRP = B // 16
NSLOT = (4 * B) // 128
ZR = 240
NZCH = SP // ZR // NS


def _bf16r(v):
    u = plsc.bitcast(v, jnp.int32)
    u = u + 0x7FFF + ((u >> 16) & 1)
    return plsc.bitcast(u & jnp.int32(-65536), jnp.float32)


def _sc_body(pos_hbm, feat_hbm, op_hbm, w2c_hbm, out_hbm, bw_hbm,
             accS, posV, featV, opV, w2cV, idxV, valV, bwV, nrmV, sem):
    c = lax.axis_index("c")
    s = lax.axis_index("s")
    lanes = lax.iota(jnp.int32, 16)
    zvec = jnp.zeros((16,), jnp.float32)

    n_pad = bw_hbm.shape[0]
    np_tile = n_pad // NS
    nchunk = np_tile // B

    pltpu.sync_copy(w2c_hbm, w2cV)

    def vloop(i, _):
        w = i * 16 + lanes
        plsc.store_scatter(valV, [w >> 10, (w >> 3) & 127, w & 7], zvec)
        return 0
    lax.fori_loop(0, NSLOT * 128 * 8 // 16, vloop, 0)

    w2c_vec = _bf16r(w2cV[pl.ds(0, 16)])
    (a00, a01, a02, a03, a10, a11, a12, a13,
     a20, a21, a22, a23) = (w2c_vec[i] for i in range(12))

    for p in range(2):
        row_lo = (c * 2 + p) * STRIP_ROWS
        pix_lo = row_lo * W

        def zloop(i, _):
            w = i * 16 + lanes
            plsc.store_scatter(nrmV, [w >> 3, w & 7], zvec)
            return 0
        lax.fori_loop(0, ZR * 8 // 16, zloop, 0)

        def zfill(k, _):
            pltpu.sync_copy(nrmV, accS.at[pl.ds((s * NZCH + k) * ZR, ZR)])
            return 0
        lax.fori_loop(0, NZCH, zfill, 0)
        plsc.subcore_barrier()

        def chunk(ch, _):
            base = s * np_tile + ch * B
            pltpu.sync_copy(pos_hbm.at[pl.ds(base, B), :], posV)
            pltpu.sync_copy(feat_hbm.at[pl.ds(base, B), :], featV)
            pltpu.sync_copy(op_hbm.at[pl.ds(base, B)], opV)

            for g in range(NGRP):
                o = g * 16
                rows16 = o + lanes
                x = _bf16r(plsc.load_gather(posV, [rows16, jnp.full((16,), 0, jnp.int32)]))
                y = _bf16r(plsc.load_gather(posV, [rows16, jnp.full((16,), 1, jnp.int32)]))
                z = _bf16r(plsc.load_gather(posV, [rows16, jnp.full((16,), 2, jnp.int32)]))
                f0 = plsc.load_gather(featV, [rows16, jnp.full((16,), 0, jnp.int32)])
                f1 = plsc.load_gather(featV, [rows16, jnp.full((16,), 1, jnp.int32)])
                f2 = plsc.load_gather(featV, [rows16, jnp.full((16,), 2, jnp.int32)])
                f3 = plsc.load_gather(featV, [rows16, jnp.full((16,), 3, jnp.int32)])
                op = opV[pl.ds(o, 16)]

                xc = x * a00 + y * a01 + z * a02 + a03
                yc = x * a10 + y * a11 + z * a12 + a13
                zc = x * a20 + y * a21 + z * a22 + a23

                valid = (zc > NEAR) & (zc < FAR)
                zs = jnp.where(valid, zc, 1.0)
                xp = (FX * xc) / zs + CX
                yp = (FY * yc) / zs + CY

                xt = xp.astype(jnp.int32)
                xtf = xt.astype(jnp.float32)
                xadj = xtf > xp
                x0i = xt - xadj.astype(jnp.int32)
                fx = xp - (xtf - xadj.astype(jnp.float32))
                yt = yp.astype(jnp.int32)
                ytf = yt.astype(jnp.float32)
                yadj = ytf > yp
                y0i = yt - yadj.astype(jnp.int32)
                fy = yp - (ytf - yadj.astype(jnp.float32))

                opv = op * valid.astype(jnp.float32)
                if p == 0:
                    @pl.when(c == 0)
                    def _():
                        bwV[pl.ds(o, 16)] = opv

                wx0 = 1.0 - fx
                wy0 = 1.0 - fy
                for k, (dx, dy, wx, wy) in enumerate(
                        ((0, 0, wx0, wy0), (1, 0, fx, wy0),
                         (0, 1, wx0, fy), (1, 1, fx, fy))):
                    cxk = x0i + dx if dx else x0i
                    cyk = y0i + dy if dy else y0i
                    instrip = ((cxk >= 0) & (cxk < W)
                               & (cyk >= row_lo) & (cyk < row_lo + STRIP_ROWS))
                    wk = wx * wy * opv * instrip.astype(jnp.float32)
                    lpix = jnp.where(instrip, (cyk - row_lo) * W + cxk, 0)
                    slot = (g * 64 + k * 16) // 128
                    roff = (g * 64 + k * 16) % 128
                    idxV[slot, pl.ds(roff, 16)] = lpix
                    vrows = roff + lanes
                    vslot = jnp.full((16,), slot, jnp.int32)
                    plsc.store_scatter(valV, [vslot, vrows, jnp.full((16,), 0, jnp.int32)], wk * f0)
                    plsc.store_scatter(valV, [vslot, vrows, jnp.full((16,), 1, jnp.int32)], wk * f1)
                    plsc.store_scatter(valV, [vslot, vrows, jnp.full((16,), 2, jnp.int32)], wk * f2)
                    plsc.store_scatter(valV, [vslot, vrows, jnp.full((16,), 3, jnp.int32)], wk * f3)
                    plsc.store_scatter(valV, [vslot, vrows, jnp.full((16,), 4, jnp.int32)], wk)

            descs = [pltpu.async_copy(valV.at[t], accS.at[idxV.at[t]], sem, add=True)
                     for t in range(NSLOT)]
            for d in descs:
                d.wait()

            if p == 0:
                @pl.when(c == 0)
                def _():
                    pltpu.sync_copy(bwV, bw_hbm.at[pl.ds(base, B)])
            return 0
        lax.fori_loop(0, nchunk, chunk, 0)
        plsc.subcore_barrier()

        def nchunk_body(k, _):
            crow = (s * NZCH + k) * ZR
            pltpu.sync_copy(accS.at[pl.ds(crow, ZR)], nrmV)

            def ngrp(gg, _):
                prow = gg * 16 + lanes
                c4 = jnp.full((16,), 4, jnp.int32)
                den = plsc.load_gather(nrmV, [prow, c4])
                alpha = jnp.minimum(jnp.maximum(den, 0.0), 1.0)
                scale = alpha / (den + 1e-8)
                for ch in range(4):
                    cc = jnp.full((16,), ch, jnp.int32)
                    nc = plsc.load_gather(nrmV, [prow, cc])
                    plsc.store_scatter(nrmV, [prow, cc], nc * scale)
                plsc.store_scatter(nrmV, [prow, c4], alpha)
                return 0
            lax.fori_loop(0, ZR // 16, ngrp, 0)

            pltpu.sync_copy(nrmV, out_hbm.at[pl.ds(pix_lo + crow, ZR)])
            return 0
        lax.fori_loop(0, NZCH, nchunk_body, 0)
        if p == 0:
            plsc.subcore_barrier()


def _splat(pos, feat, op, w2c_flat, n_pad):
    mesh = plsc.VectorSubcoreMesh(core_axis_name="c", subcore_axis_name="s")
    run = functools.partial(
        pl.kernel,
        out_type=(jax.ShapeDtypeStruct((H * W, 8), jnp.float32),
                  jax.ShapeDtypeStruct((n_pad,), jnp.float32)),
        mesh=mesh,
        compiler_params=pltpu.CompilerParams(needs_layout_passes=False, use_tc_tiling_on_sc=False),
        scratch_types=[
            pltpu.VMEM_SHARED((SP, 8), jnp.float32),
            pltpu.VMEM((B, 3), jnp.float32),
            pltpu.VMEM((B, 4), jnp.float32),
            pltpu.VMEM((B,), jnp.float32),
            pltpu.VMEM((16,), jnp.float32),
            pltpu.VMEM((NSLOT, 128), jnp.int32),
            pltpu.VMEM((NSLOT, 128, 8), jnp.float32),
            pltpu.VMEM((B,), jnp.float32),
            pltpu.VMEM((ZR, 8), jnp.float32),
            pltpu.SemaphoreType.DMA,
        ],
    )(_sc_body)
    return run(pos, feat, op, w2c_flat)


def kernel(positions, features, opacities, w2c, cam_position, mode):
    n = positions.shape[0]
    cpp = NS * B
    n_pad = ((n + cpp - 1) // cpp) * cpp
    pad = n_pad - n
    pos = jnp.pad(positions, ((0, pad), (0, 0)))
    feat = jnp.pad(features, ((0, pad), (0, 0)))
    op = jnp.pad(opacities[:, 0], (0, pad))
    acc, bw = _splat(pos, feat, op, w2c.reshape(16), n_pad)
    image = acc[:, :4].reshape(H, W, 4).transpose(2, 0, 1)
    alpha = acc[:, 4].reshape(H, W)
    return image, alpha, bw[:n]
